# baseline (speedup 1.0000x reference)
import functools

import jax
import jax.numpy as jnp
from jax import lax
from jax.experimental import pallas as pl
from jax.experimental.pallas import tpu as pltpu
from jax.experimental.pallas import tpu_sc as plsc

N = 10000
E = 320000
D = 128
H = 128
C = 40

NC = 2
NS = 16
NW = NC * NS
EPW = E // NW
K = 80
NCHUNK = EPW // K
NBUF = 5
NBS = 3
REM = NCHUNK % NBS
NP = 10240
RPT = NP // NS
GR = NP // H

_MESH = dict(core_axis_name="c", subcore_axis_name="s", num_cores=NC,
             num_subcores=NS)


def _deg_body(src_hbm, dst_hbm, z_hbm, ones_hbm, out_hbm,
              e0, e1, e2, e3, e4, ones_v, acc, *sems):
    c = lax.axis_index("c")
    s = lax.axis_index("s")
    wid = s * NC + c
    ebase = wid * EPW
    row0 = s * RPT
    eidx = (e0, e1, e2, e3, e4)
    dsem = sems[0:NBUF]
    ssem = sems[NBUF:2 * NBUF]
    pltpu.sync_copy(ones_hbm, ones_v)
    for which, e_hbm in ((0, src_hbm), (1, dst_hbm)):
        pltpu.sync_copy(z_hbm.at[pl.ds(row0, RPT), :],
                        acc.at[pl.ds(row0, RPT), :])
        plsc.subcore_barrier()

        def start(b, ch):
            pltpu.async_copy(e_hbm.at[pl.ds(ebase + ch * K, K)], eidx[b],
                             dsem[b])

        def wait_in(b):
            pltpu.make_async_copy(e_hbm.at[pl.ds(0, K)], eidx[b],
                                  dsem[b]).wait()

        for b in range(NBUF):
            start(b, b)

        @pl.loop(0, NCHUNK - NBUF, step=NBUF)
        def _(g):
            descs = []
            for b in range(NBUF):
                wait_in(b)
                descs.append(pltpu.async_copy(ones_v, acc.at[eidx[b]],
                                              ssem[b], add=True))
            for b in range(NBUF):
                descs[b].wait()
                start(b, g + NBUF + b)

        descs = []
        for b in range(NBUF):
            wait_in(b)
            descs.append(pltpu.async_copy(ones_v, acc.at[eidx[b]],
                                          ssem[b], add=True))
        for d in descs:
            d.wait()
        plsc.subcore_barrier()
        pltpu.sync_copy(acc.at[pl.ds(row0, RPT), :],
                        out_hbm.at[which, c, pl.ds(row0, RPT), :])
        plsc.subcore_barrier()


def _degrees(src, dst, z128, ones):
    f = pl.kernel(
        _deg_body,
        out_type=jax.ShapeDtypeStruct((2, NC, NP, H), jnp.float32),
        mesh=plsc.VectorSubcoreMesh(**_MESH),
        scratch_types=(
            [pltpu.VMEM((K,), jnp.int32) for _ in range(NBUF)]
            + [pltpu.VMEM((K, H), jnp.float32)]
            + [pltpu.VMEM_SHARED((NP, H), jnp.float32)]
            + [pltpu.SemaphoreType.DMA for _ in range(2 * NBUF)]
        ),
    )
    return f(src, dst, z128, ones)


def _spmm_body(h_hbm, src_hbm, dst_hbm, z_hbm, out_hbm,
               sidx, d0, d1, d2, r0b, r1b, r2b, acc, *sems):
    c = lax.axis_index("c")
    s = lax.axis_index("s")
    wid = s * NC + c
    ebase = wid * EPW
    row0 = s * RPT
    didx = (d0, d1, d2)
    rows = (r0b, r1b, r2b)
    dsem = sems[0:NBS]
    gsem = sems[NBS:2 * NBS]
    ssem = sems[2 * NBS:3 * NBS]

    pltpu.sync_copy(z_hbm.at[pl.ds(row0, RPT), :],
                    acc.at[pl.ds(row0, RPT), :])
    pltpu.sync_copy(src_hbm.at[pl.ds(ebase, EPW)], sidx)
    plsc.subcore_barrier()

    def start(b, ch):
        pltpu.async_copy(dst_hbm.at[pl.ds(ebase + ch * K, K)], didx[b],
                         dsem[b])
        pltpu.async_copy(h_hbm.at[sidx.at[pl.ds(ch * K, K)]], rows[b],
                         gsem[b])

    def wait_in(b):
        pltpu.make_async_copy(dst_hbm.at[pl.ds(0, K)], didx[b],
                              dsem[b]).wait()
        pltpu.make_async_copy(h_hbm.at[sidx.at[pl.ds(0, K)]], rows[b],
                              gsem[b]).wait()

    def scatter(b):
        return pltpu.async_copy(rows[b], acc.at[didx[b]], ssem[b],
                                add=True)

    for b in range(NBS):
        start(b, b)

    @pl.loop(0, NCHUNK - REM - NBS, step=NBS)
    def _(g):
        descs = []
        for b in range(NBS):
            wait_in(b)
            descs.append(scatter(b))
        for b in range(NBS):
            descs[b].wait()
            ch2 = g + NBS + b

            @pl.when(ch2 < NCHUNK)
            def _():
                start(b, ch2)

    descs = []
    for b in range(NBS):
        wait_in(b)
        descs.append(scatter(b))
    for b in range(NBS):
        descs[b].wait()
        if b < REM:
            start(b, NCHUNK - REM + b)
    for b in range(REM):
        wait_in(b)
        scatter(b).wait()
    plsc.subcore_barrier()
    pltpu.sync_copy(acc.at[pl.ds(row0, RPT), :],
                    out_hbm.at[c, pl.ds(row0, RPT), :])


def _spmm(h, src, dst, z128):
    f = pl.kernel(
        _spmm_body,
        out_type=jax.ShapeDtypeStruct((NC, NP, H), jnp.float32),
        mesh=plsc.VectorSubcoreMesh(**_MESH),
        scratch_types=(
            [pltpu.VMEM((EPW,), jnp.int32)]
            + [pltpu.VMEM((K,), jnp.int32) for _ in range(NBS)]
            + [pltpu.VMEM((K, H), jnp.float32) for _ in range(NBS)]
            + [pltpu.VMEM_SHARED((NP, H), jnp.float32)]
            + [pltpu.SemaphoreType.DMA for _ in range(3 * NBS)]
        ),
    )
    return f(h, src, dst, z128)


def _pre_body(feat_ref, degs_ref, h0_ref, osc_ref, isc_ref):
    scnt = degs_ref[0, 0, 0:N] + degs_ref[0, 1, 0:N]
    dcnt = degs_ref[1, 0, 0:N] + degs_ref[1, 1, 0:N]
    osc = lax.rsqrt(jnp.maximum(scnt, 1.0))
    isc = lax.rsqrt(jnp.maximum(dcnt, 1.0))
    h0_ref[...] = feat_ref[...] * osc
    osc_ref[...] = osc
    isc_ref[...] = isc


def _pre(feat, degs):
    return pl.pallas_call(
        _pre_body,
        out_shape=(
            jax.ShapeDtypeStruct((N, D), jnp.float32),
            jax.ShapeDtypeStruct((N, H), jnp.float32),
            jax.ShapeDtypeStruct((N, H), jnp.float32),
        ),
    )(feat, degs)


def _post_body(p_ref, isc_ref, osc_ref, wt_ref, g_ref, bt_ref, out_ref):
    r = (p_ref[0, 0:N] + p_ref[1, 0:N]) * isc_ref[...]
    y = jnp.dot(r, wt_ref[...], preferred_element_type=jnp.float32)
    mu = jnp.mean(y, axis=0, keepdims=True)
    yc = y - mu
    var = jnp.mean(yc * yc, axis=0, keepdims=True)
    h = jnp.maximum(yc * lax.rsqrt(var + 1e-5) * g_ref[...] + bt_ref[...],
                    0.0)
    out_ref[...] = h * osc_ref[...]


def _post(p, isc, osc, wt, g, bt):
    return pl.pallas_call(
        _post_body,
        out_shape=jax.ShapeDtypeStruct((N, H), jnp.float32),
    )(p, isc, osc, wt, g, bt)


def _final_body(p_ref, isc_ref, w2t_ref, b2_ref, out_ref):
    r = (p_ref[0, 0:N] + p_ref[1, 0:N]) * isc_ref[...]
    out_ref[...] = (jnp.dot(r, w2t_ref[...],
                            preferred_element_type=jnp.float32)
                    + b2_ref[...])


def _final(p, isc, w2t, b2):
    return pl.pallas_call(
        _final_body,
        out_shape=jax.ShapeDtypeStruct((N, C), jnp.float32),
    )(p, isc, w2t, b2)


def kernel(feat, edge_index, W0, W1, W2, b2, g0, bt0, g1, bt1):
    src = edge_index[0]
    dst = edge_index[1]
    z128 = jnp.zeros((NP, H), jnp.float32)
    ones = jnp.ones((K, H), jnp.float32)

    degs = _degrees(src, dst, z128, ones)
    h0, osc, isc = _pre(feat, degs)
    p = _spmm(h0, src, dst, z128)
    h1 = _post(p, isc, osc, W0.T, g0[None], bt0[None])
    p = _spmm(h1, src, dst, z128)
    h2 = _post(p, isc, osc, W1.T, g1[None], bt1[None])
    p = _spmm(h2, src, dst, z128)
    return _final(p, isc, W2.T, b2[None])


# speedup vs baseline: 1.0403x; 1.0403x over previous
import functools

import jax
import jax.numpy as jnp
from jax import lax
from jax.experimental import pallas as pl
from jax.experimental.pallas import tpu as pltpu
from jax.experimental.pallas import tpu_sc as plsc

N = 10000
E = 320000
D = 128
H = 128
C = 40

NC = 2
NS = 16
NW = NC * NS
EPW = E // NW
KD = 80
NCD = EPW // KD
NBD = 5
KS = 40
NCS = EPW // KS
NBS = 5
REM = NCS % NBS
NP = 10240
RPT = NP // NS
GR = NP // H

_MESH = dict(core_axis_name="c", subcore_axis_name="s", num_cores=NC,
             num_subcores=NS)


def _deg_body(src_hbm, dst_hbm, z_hbm, ones_hbm, out_hbm,
              e0, e1, e2, e3, e4, ones_v, acc, *sems):
    c = lax.axis_index("c")
    s = lax.axis_index("s")
    wid = s * NC + c
    ebase = wid * EPW
    row0 = s * RPT
    eidx = (e0, e1, e2, e3, e4)
    dsem = sems[0:NBD]
    ssem = sems[NBD:2 * NBD]
    pltpu.sync_copy(ones_hbm, ones_v)
    for which, e_hbm in ((0, src_hbm), (1, dst_hbm)):
        pltpu.sync_copy(z_hbm.at[pl.ds(row0, RPT), :],
                        acc.at[pl.ds(row0, RPT), :])
        plsc.subcore_barrier()

        def start(b, ch):
            pltpu.async_copy(e_hbm.at[pl.ds(ebase + ch * KD, KD)], eidx[b],
                             dsem[b])

        def wait_in(b):
            pltpu.make_async_copy(e_hbm.at[pl.ds(0, KD)], eidx[b],
                                  dsem[b]).wait()

        for b in range(NBD):
            start(b, b)

        @pl.loop(0, NCD - NBD, step=NBD)
        def _(g):
            descs = []
            for b in range(NBD):
                wait_in(b)
                descs.append(pltpu.async_copy(ones_v, acc.at[eidx[b]],
                                              ssem[b], add=True))
            for b in range(NBD):
                descs[b].wait()
                start(b, g + NBD + b)

        descs = []
        for b in range(NBD):
            wait_in(b)
            descs.append(pltpu.async_copy(ones_v, acc.at[eidx[b]],
                                          ssem[b], add=True))
        for d in descs:
            d.wait()
        plsc.subcore_barrier()
        pltpu.sync_copy(acc.at[pl.ds(row0, RPT), :],
                        out_hbm.at[which, c, pl.ds(row0, RPT), :])
        plsc.subcore_barrier()


def _degrees(src, dst, z128, ones):
    f = pl.kernel(
        _deg_body,
        out_type=jax.ShapeDtypeStruct((2, NC, NP, H), jnp.float32),
        mesh=plsc.VectorSubcoreMesh(**_MESH),
        scratch_types=(
            [pltpu.VMEM((KD,), jnp.int32) for _ in range(NBD)]
            + [pltpu.VMEM((KD, H), jnp.float32)]
            + [pltpu.VMEM_SHARED((NP, H), jnp.float32)]
            + [pltpu.SemaphoreType.DMA for _ in range(2 * NBD)]
        ),
    )
    return f(src, dst, z128, ones)


def _spmm_body(h_hbm, src_hbm, dst_hbm, z_hbm, out_hbm,
               sidx, d0, d1, d2, d3, d4, r0b, r1b, r2b, r3b, r4b, acc,
               *sems):
    c = lax.axis_index("c")
    s = lax.axis_index("s")
    wid = s * NC + c
    ebase = wid * EPW
    row0 = s * RPT
    didx = (d0, d1, d2, d3, d4)
    rows = (r0b, r1b, r2b, r3b, r4b)
    dsem = sems[0:NBS]
    gsem = sems[NBS:2 * NBS]
    ssem = sems[2 * NBS:3 * NBS]

    pltpu.sync_copy(z_hbm.at[pl.ds(row0, RPT), :],
                    acc.at[pl.ds(row0, RPT), :])
    pltpu.sync_copy(src_hbm.at[pl.ds(ebase, EPW)], sidx)
    plsc.subcore_barrier()

    def start(b, ch):
        pltpu.async_copy(dst_hbm.at[pl.ds(ebase + ch * KS, KS)], didx[b],
                         dsem[b])
        pltpu.async_copy(h_hbm.at[sidx.at[pl.ds(ch * KS, KS)]], rows[b],
                         gsem[b])

    def wait_in(b):
        pltpu.make_async_copy(dst_hbm.at[pl.ds(0, KS)], didx[b],
                              dsem[b]).wait()
        pltpu.make_async_copy(h_hbm.at[sidx.at[pl.ds(0, KS)]], rows[b],
                              gsem[b]).wait()

    def scatter(b):
        return pltpu.async_copy(rows[b], acc.at[didx[b]], ssem[b],
                                add=True)

    for b in range(NBS):
        start(b, b)

    @pl.loop(0, NCS - REM - NBS, step=NBS)
    def _(g):
        descs = []
        for b in range(NBS):
            wait_in(b)
            descs.append(scatter(b))
        for b in range(NBS):
            descs[b].wait()
            ch2 = g + NBS + b

            @pl.when(ch2 < NCS)
            def _():
                start(b, ch2)

    descs = []
    for b in range(NBS):
        wait_in(b)
        descs.append(scatter(b))
    for b in range(NBS):
        descs[b].wait()
        if b < REM:
            start(b, NCS - REM + b)
    for b in range(REM):
        wait_in(b)
        scatter(b).wait()
    plsc.subcore_barrier()
    pltpu.sync_copy(acc.at[pl.ds(row0, RPT), :],
                    out_hbm.at[c, pl.ds(row0, RPT), :])


def _spmm(h, src, dst, z128):
    f = pl.kernel(
        _spmm_body,
        out_type=jax.ShapeDtypeStruct((NC, NP, H), jnp.float32),
        mesh=plsc.VectorSubcoreMesh(**_MESH),
        scratch_types=(
            [pltpu.VMEM((EPW,), jnp.int32)]
            + [pltpu.VMEM((KS,), jnp.int32) for _ in range(NBS)]
            + [pltpu.VMEM((KS, H), jnp.float32) for _ in range(NBS)]
            + [pltpu.VMEM_SHARED((NP, H), jnp.float32)]
            + [pltpu.SemaphoreType.DMA for _ in range(3 * NBS)]
        ),
    )
    return f(h, src, dst, z128)


def _pre_body(feat_ref, degs_ref, h0_ref, osc_ref, isc_ref):
    scnt = degs_ref[0, 0, 0:N] + degs_ref[0, 1, 0:N]
    dcnt = degs_ref[1, 0, 0:N] + degs_ref[1, 1, 0:N]
    osc = lax.rsqrt(jnp.maximum(scnt, 1.0))
    isc = lax.rsqrt(jnp.maximum(dcnt, 1.0))
    h0_ref[...] = feat_ref[...] * osc
    osc_ref[...] = osc
    isc_ref[...] = isc


def _pre(feat, degs):
    return pl.pallas_call(
        _pre_body,
        out_shape=(
            jax.ShapeDtypeStruct((N, D), jnp.float32),
            jax.ShapeDtypeStruct((N, H), jnp.float32),
            jax.ShapeDtypeStruct((N, H), jnp.float32),
        ),
    )(feat, degs)


def _post_body(p_ref, isc_ref, osc_ref, wt_ref, g_ref, bt_ref, out_ref):
    r = (p_ref[0, 0:N] + p_ref[1, 0:N]) * isc_ref[...]
    y = jnp.dot(r, wt_ref[...], preferred_element_type=jnp.float32)
    mu = jnp.mean(y, axis=0, keepdims=True)
    yc = y - mu
    var = jnp.mean(yc * yc, axis=0, keepdims=True)
    h = jnp.maximum(yc * lax.rsqrt(var + 1e-5) * g_ref[...] + bt_ref[...],
                    0.0)
    out_ref[...] = h * osc_ref[...]


def _post(p, isc, osc, wt, g, bt):
    return pl.pallas_call(
        _post_body,
        out_shape=jax.ShapeDtypeStruct((N, H), jnp.float32),
    )(p, isc, osc, wt, g, bt)


def _final_body(p_ref, isc_ref, w2t_ref, b2_ref, out_ref):
    r = (p_ref[0, 0:N] + p_ref[1, 0:N]) * isc_ref[...]
    out_ref[...] = (jnp.dot(r, w2t_ref[...],
                            preferred_element_type=jnp.float32)
                    + b2_ref[...])


def _final(p, isc, w2t, b2):
    return pl.pallas_call(
        _final_body,
        out_shape=jax.ShapeDtypeStruct((N, C), jnp.float32),
    )(p, isc, w2t, b2)


def kernel(feat, edge_index, W0, W1, W2, b2, g0, bt0, g1, bt1):
    src = edge_index[0]
    dst = edge_index[1]
    z128 = jnp.zeros((NP, H), jnp.float32)
    ones = jnp.ones((KD, H), jnp.float32)

    degs = _degrees(src, dst, z128, ones)
    h0, osc, isc = _pre(feat, degs)
    p = _spmm(h0, src, dst, z128)
    h1 = _post(p, isc, osc, W0.T, g0[None], bt0[None])
    p = _spmm(h1, src, dst, z128)
    h2 = _post(p, isc, osc, W1.T, g1[None], bt1[None])
    p = _spmm(h2, src, dst, z128)
    return _final(p, isc, W2.T, b2[None])


# speedup vs baseline: 1.0602x; 1.0191x over previous
import functools

import jax
import jax.numpy as jnp
from jax import lax
from jax.experimental import pallas as pl
from jax.experimental.pallas import tpu as pltpu
from jax.experimental.pallas import tpu_sc as plsc

N = 10000
E = 320000
D = 128
H = 128
C = 40

NC = 2
NS = 16
NW = NC * NS
EPW = E // NW
KD = 80
NCD = EPW // KD
NBD = 5
KS = 40
NCS = EPW // KS
NBS = 6
REM = NCS % NBS
NP = 10240
RPT = NP // NS
GR = NP // H

_MESH = dict(core_axis_name="c", subcore_axis_name="s", num_cores=NC,
             num_subcores=NS)


def _deg_body(src_hbm, dst_hbm, z_hbm, ones_hbm, out_hbm,
              e0, e1, e2, e3, e4, ones_v, acc, *sems):
    c = lax.axis_index("c")
    s = lax.axis_index("s")
    wid = s * NC + c
    ebase = wid * EPW
    row0 = s * RPT
    eidx = (e0, e1, e2, e3, e4)
    dsem = sems[0:NBD]
    ssem = sems[NBD:2 * NBD]
    pltpu.sync_copy(ones_hbm, ones_v)
    for which, e_hbm in ((0, src_hbm), (1, dst_hbm)):
        pltpu.sync_copy(z_hbm.at[pl.ds(row0, RPT), :],
                        acc.at[pl.ds(row0, RPT), :])
        plsc.subcore_barrier()

        def start(b, ch):
            pltpu.async_copy(e_hbm.at[pl.ds(ebase + ch * KD, KD)], eidx[b],
                             dsem[b])

        def wait_in(b):
            pltpu.make_async_copy(e_hbm.at[pl.ds(0, KD)], eidx[b],
                                  dsem[b]).wait()

        for b in range(NBD):
            start(b, b)

        @pl.loop(0, NCD - NBD, step=NBD)
        def _(g):
            descs = []
            for b in range(NBD):
                wait_in(b)
                descs.append(pltpu.async_copy(ones_v, acc.at[eidx[b]],
                                              ssem[b], add=True))
            for b in range(NBD):
                descs[b].wait()
                start(b, g + NBD + b)

        descs = []
        for b in range(NBD):
            wait_in(b)
            descs.append(pltpu.async_copy(ones_v, acc.at[eidx[b]],
                                          ssem[b], add=True))
        for d in descs:
            d.wait()
        plsc.subcore_barrier()
        pltpu.sync_copy(acc.at[pl.ds(row0, RPT), :],
                        out_hbm.at[which, c, pl.ds(row0, RPT), :])
        plsc.subcore_barrier()


def _degrees(src, dst, z128, ones):
    f = pl.kernel(
        _deg_body,
        out_type=jax.ShapeDtypeStruct((2, NC, NP, H), jnp.float32),
        mesh=plsc.VectorSubcoreMesh(**_MESH),
        scratch_types=(
            [pltpu.VMEM((KD,), jnp.int32) for _ in range(NBD)]
            + [pltpu.VMEM((KD, H), jnp.float32)]
            + [pltpu.VMEM_SHARED((NP, H), jnp.float32)]
            + [pltpu.SemaphoreType.DMA for _ in range(2 * NBD)]
        ),
    )
    return f(src, dst, z128, ones)


def _spmm_body(h_hbm, src_hbm, dst_hbm, z_hbm, out_hbm,
               sidx, d0, d1, d2, d3, d4, d5, r0b, r1b, r2b, r3b, r4b, r5b,
               acc, *sems):
    c = lax.axis_index("c")
    s = lax.axis_index("s")
    wid = s * NC + c
    ebase = wid * EPW
    row0 = s * RPT
    didx = (d0, d1, d2, d3, d4, d5)
    rows = (r0b, r1b, r2b, r3b, r4b, r5b)
    dsem = sems[0:NBS]
    gsem = sems[NBS:2 * NBS]
    ssem = sems[2 * NBS:3 * NBS]
    isem = sems[3 * NBS]

    sdesc = pltpu.async_copy(src_hbm.at[pl.ds(ebase, EPW)], sidx, isem)
    pltpu.sync_copy(z_hbm.at[pl.ds(row0, RPT), :],
                    acc.at[pl.ds(row0, RPT), :])
    sdesc.wait()

    def start(b, ch):
        pltpu.async_copy(dst_hbm.at[pl.ds(ebase + ch * KS, KS)], didx[b],
                         dsem[b])
        pltpu.async_copy(h_hbm.at[sidx.at[pl.ds(ch * KS, KS)]], rows[b],
                         gsem[b])

    def wait_in(b):
        pltpu.make_async_copy(dst_hbm.at[pl.ds(0, KS)], didx[b],
                              dsem[b]).wait()
        pltpu.make_async_copy(h_hbm.at[sidx.at[pl.ds(0, KS)]], rows[b],
                              gsem[b]).wait()

    def scatter(b):
        return pltpu.async_copy(rows[b], acc.at[didx[b]], ssem[b],
                                add=True)

    for b in range(NBS):
        start(b, b)
    plsc.subcore_barrier()

    @pl.loop(0, NCS - REM - NBS, step=NBS)
    def _(g):
        descs = []
        for b in range(NBS):
            wait_in(b)
            descs.append(scatter(b))
        for b in range(NBS):
            descs[b].wait()
            ch2 = g + NBS + b

            @pl.when(ch2 < NCS)
            def _():
                start(b, ch2)

    descs = []
    for b in range(NBS):
        wait_in(b)
        descs.append(scatter(b))
    for b in range(NBS):
        descs[b].wait()
        if b < REM:
            start(b, NCS - REM + b)
    for b in range(REM):
        wait_in(b)
        scatter(b).wait()
    plsc.subcore_barrier()
    pltpu.sync_copy(acc.at[pl.ds(row0, RPT), :],
                    out_hbm.at[c, pl.ds(row0, RPT), :])


def _spmm(h, src, dst, z128):
    f = pl.kernel(
        _spmm_body,
        out_type=jax.ShapeDtypeStruct((NC, NP, H), jnp.float32),
        mesh=plsc.VectorSubcoreMesh(**_MESH),
        scratch_types=(
            [pltpu.VMEM((EPW,), jnp.int32)]
            + [pltpu.VMEM((KS,), jnp.int32) for _ in range(NBS)]
            + [pltpu.VMEM((KS, H), jnp.float32) for _ in range(NBS)]
            + [pltpu.VMEM_SHARED((NP, H), jnp.float32)]
            + [pltpu.SemaphoreType.DMA for _ in range(3 * NBS + 1)]
        ),
    )
    return f(h, src, dst, z128)


def _pre_body(feat_ref, degs_ref, h0_ref, osc_ref, isc_ref):
    scnt = degs_ref[0, 0, 0:N] + degs_ref[0, 1, 0:N]
    dcnt = degs_ref[1, 0, 0:N] + degs_ref[1, 1, 0:N]
    osc = lax.rsqrt(jnp.maximum(scnt, 1.0))
    isc = lax.rsqrt(jnp.maximum(dcnt, 1.0))
    h0_ref[...] = feat_ref[...] * osc
    osc_ref[...] = osc
    isc_ref[...] = isc


def _pre(feat, degs):
    return pl.pallas_call(
        _pre_body,
        out_shape=(
            jax.ShapeDtypeStruct((N, D), jnp.float32),
            jax.ShapeDtypeStruct((N, H), jnp.float32),
            jax.ShapeDtypeStruct((N, H), jnp.float32),
        ),
    )(feat, degs)


def _post_body(p_ref, isc_ref, osc_ref, wt_ref, g_ref, bt_ref, out_ref):
    r = (p_ref[0, 0:N] + p_ref[1, 0:N]) * isc_ref[...]
    y = jnp.dot(r, wt_ref[...], preferred_element_type=jnp.float32)
    mu = jnp.mean(y, axis=0, keepdims=True)
    yc = y - mu
    var = jnp.mean(yc * yc, axis=0, keepdims=True)
    h = jnp.maximum(yc * lax.rsqrt(var + 1e-5) * g_ref[...] + bt_ref[...],
                    0.0)
    out_ref[...] = h * osc_ref[...]


def _post(p, isc, osc, wt, g, bt):
    return pl.pallas_call(
        _post_body,
        out_shape=jax.ShapeDtypeStruct((N, H), jnp.float32),
    )(p, isc, osc, wt, g, bt)


def _final_body(p_ref, isc_ref, w2t_ref, b2_ref, out_ref):
    r = (p_ref[0, 0:N] + p_ref[1, 0:N]) * isc_ref[...]
    out_ref[...] = (jnp.dot(r, w2t_ref[...],
                            preferred_element_type=jnp.float32)
                    + b2_ref[...])


def _final(p, isc, w2t, b2):
    return pl.pallas_call(
        _final_body,
        out_shape=jax.ShapeDtypeStruct((N, C), jnp.float32),
    )(p, isc, w2t, b2)


def kernel(feat, edge_index, W0, W1, W2, b2, g0, bt0, g1, bt1):
    src = edge_index[0]
    dst = edge_index[1]
    z128 = jnp.zeros((NP, H), jnp.float32)
    ones = jnp.ones((KD, H), jnp.float32)

    degs = _degrees(src, dst, z128, ones)
    h0, osc, isc = _pre(feat, degs)
    p = _spmm(h0, src, dst, z128)
    h1 = _post(p, isc, osc, W0.T, g0[None], bt0[None])
    p = _spmm(h1, src, dst, z128)
    h2 = _post(p, isc, osc, W1.T, g1[None], bt1[None])
    p = _spmm(h2, src, dst, z128)
    return _final(p, isc, W2.T, b2[None])


# speedup vs baseline: 1.0929x; 1.0308x over previous
import functools

import jax
import jax.numpy as jnp
from jax import lax
from jax.experimental import pallas as pl
from jax.experimental.pallas import tpu as pltpu
from jax.experimental.pallas import tpu_sc as plsc

N = 10000
E = 320000
D = 128
H = 128
C = 40

NC = 2
NS = 16
NW = NC * NS
EPW = E // NW
KD = 80
EPW2 = E // NS
NCD = EPW2 // KD
NBD = 5
KS = 40
NCS = EPW // KS
NBS = 6
REM = NCS % NBS
NP = 10240
RPT = NP // NS
GR = NP // H

_MESH = dict(core_axis_name="c", subcore_axis_name="s", num_cores=NC,
             num_subcores=NS)


def _deg_body(src_hbm, dst_hbm, z_hbm, ones_hbm, out_hbm,
              e0, e1, e2, e3, e4, ones_v, acc, *sems):
    c = lax.axis_index("c")
    s = lax.axis_index("s")
    row0 = s * RPT
    ebase = s * EPW2
    eidx = (e0, e1, e2, e3, e4)
    dsem = sems[0:NBD]
    ssem = sems[NBD:2 * NBD]
    pltpu.sync_copy(ones_hbm, ones_v)

    def one_count(e_hbm, which):
        pltpu.sync_copy(z_hbm.at[pl.ds(row0, RPT), :],
                        acc.at[pl.ds(row0, RPT), :])

        def start(b, ch):
            pltpu.async_copy(e_hbm.at[pl.ds(ebase + ch * KD, KD)], eidx[b],
                             dsem[b])

        def wait_in(b):
            pltpu.make_async_copy(e_hbm.at[pl.ds(0, KD)], eidx[b],
                                  dsem[b]).wait()

        for b in range(NBD):
            start(b, b)
        plsc.subcore_barrier()

        @pl.loop(0, NCD - NBD, step=NBD)
        def _(g):
            descs = []
            for b in range(NBD):
                wait_in(b)
                descs.append(pltpu.async_copy(ones_v, acc.at[eidx[b]],
                                              ssem[b], add=True))
            for b in range(NBD):
                descs[b].wait()
                start(b, g + NBD + b)

        descs = []
        for b in range(NBD):
            wait_in(b)
            descs.append(pltpu.async_copy(ones_v, acc.at[eidx[b]],
                                          ssem[b], add=True))
        for d in descs:
            d.wait()
        plsc.subcore_barrier()
        pltpu.sync_copy(acc.at[pl.ds(row0, RPT), :],
                        out_hbm.at[which, pl.ds(row0, RPT), :])

    @pl.when(c == 0)
    def _():
        one_count(src_hbm, 0)

    @pl.when(c == 1)
    def _():
        one_count(dst_hbm, 1)


def _degrees(src, dst, z128, ones):
    f = pl.kernel(
        _deg_body,
        out_type=jax.ShapeDtypeStruct((2, NP, H), jnp.float32),
        mesh=plsc.VectorSubcoreMesh(**_MESH),
        scratch_types=(
            [pltpu.VMEM((KD,), jnp.int32) for _ in range(NBD)]
            + [pltpu.VMEM((KD, H), jnp.float32)]
            + [pltpu.VMEM_SHARED((NP, H), jnp.float32)]
            + [pltpu.SemaphoreType.DMA for _ in range(2 * NBD)]
        ),
    )
    return f(src, dst, z128, ones)


def _spmm_body(h_hbm, src_hbm, dst_hbm, z_hbm, out_hbm,
               sidx, d0, d1, d2, d3, d4, d5, r0b, r1b, r2b, r3b, r4b, r5b,
               acc, *sems):
    c = lax.axis_index("c")
    s = lax.axis_index("s")
    wid = s * NC + c
    ebase = wid * EPW
    row0 = s * RPT
    didx = (d0, d1, d2, d3, d4, d5)
    rows = (r0b, r1b, r2b, r3b, r4b, r5b)
    dsem = sems[0:NBS]
    gsem = sems[NBS:2 * NBS]
    ssem = sems[2 * NBS:3 * NBS]
    isem = sems[3 * NBS]

    sdesc = pltpu.async_copy(src_hbm.at[pl.ds(ebase, EPW)], sidx, isem)
    pltpu.sync_copy(z_hbm.at[pl.ds(row0, RPT), :],
                    acc.at[pl.ds(row0, RPT), :])
    sdesc.wait()

    def start(b, ch):
        pltpu.async_copy(dst_hbm.at[pl.ds(ebase + ch * KS, KS)], didx[b],
                         dsem[b])
        pltpu.async_copy(h_hbm.at[sidx.at[pl.ds(ch * KS, KS)]], rows[b],
                         gsem[b])

    def wait_in(b):
        pltpu.make_async_copy(dst_hbm.at[pl.ds(0, KS)], didx[b],
                              dsem[b]).wait()
        pltpu.make_async_copy(h_hbm.at[sidx.at[pl.ds(0, KS)]], rows[b],
                              gsem[b]).wait()

    def scatter(b):
        return pltpu.async_copy(rows[b], acc.at[didx[b]], ssem[b],
                                add=True)

    for b in range(NBS):
        start(b, b)
    plsc.subcore_barrier()

    @pl.loop(0, NCS - REM - NBS, step=NBS)
    def _(g):
        descs = []
        for b in range(NBS):
            wait_in(b)
            descs.append(scatter(b))
        for b in range(NBS):
            descs[b].wait()
            ch2 = g + NBS + b

            @pl.when(ch2 < NCS)
            def _():
                start(b, ch2)

    descs = []
    for b in range(NBS):
        wait_in(b)
        descs.append(scatter(b))
    for b in range(NBS):
        descs[b].wait()
        if b < REM:
            start(b, NCS - REM + b)
    for b in range(REM):
        wait_in(b)
        scatter(b).wait()
    plsc.subcore_barrier()
    pltpu.sync_copy(acc.at[pl.ds(row0, RPT), :],
                    out_hbm.at[c, pl.ds(row0, RPT), :])


def _spmm(h, src, dst, z128):
    f = pl.kernel(
        _spmm_body,
        out_type=jax.ShapeDtypeStruct((NC, NP, H), jnp.float32),
        mesh=plsc.VectorSubcoreMesh(**_MESH),
        scratch_types=(
            [pltpu.VMEM((EPW,), jnp.int32)]
            + [pltpu.VMEM((KS,), jnp.int32) for _ in range(NBS)]
            + [pltpu.VMEM((KS, H), jnp.float32) for _ in range(NBS)]
            + [pltpu.VMEM_SHARED((NP, H), jnp.float32)]
            + [pltpu.SemaphoreType.DMA for _ in range(3 * NBS + 1)]
        ),
    )
    return f(h, src, dst, z128)


def _pre_body(feat_ref, degs_ref, h0_ref, osc_ref, isc_ref):
    scnt = degs_ref[0, 0:N]
    dcnt = degs_ref[1, 0:N]
    osc = lax.rsqrt(jnp.maximum(scnt, 1.0))
    isc = lax.rsqrt(jnp.maximum(dcnt, 1.0))
    h0_ref[...] = feat_ref[...] * osc
    osc_ref[...] = osc
    isc_ref[...] = isc


def _pre(feat, degs):
    return pl.pallas_call(
        _pre_body,
        out_shape=(
            jax.ShapeDtypeStruct((N, D), jnp.float32),
            jax.ShapeDtypeStruct((N, H), jnp.float32),
            jax.ShapeDtypeStruct((N, H), jnp.float32),
        ),
    )(feat, degs)


def _post_body(p_ref, isc_ref, osc_ref, wt_ref, g_ref, bt_ref, out_ref):
    r = (p_ref[0, 0:N] + p_ref[1, 0:N]) * isc_ref[...]
    y = jnp.dot(r, wt_ref[...], preferred_element_type=jnp.float32)
    mu = jnp.mean(y, axis=0, keepdims=True)
    yc = y - mu
    var = jnp.mean(yc * yc, axis=0, keepdims=True)
    h = jnp.maximum(yc * lax.rsqrt(var + 1e-5) * g_ref[...] + bt_ref[...],
                    0.0)
    out_ref[...] = h * osc_ref[...]


def _post(p, isc, osc, wt, g, bt):
    return pl.pallas_call(
        _post_body,
        out_shape=jax.ShapeDtypeStruct((N, H), jnp.float32),
    )(p, isc, osc, wt, g, bt)


def _final_body(p_ref, isc_ref, w2t_ref, b2_ref, out_ref):
    r = (p_ref[0, 0:N] + p_ref[1, 0:N]) * isc_ref[...]
    out_ref[...] = (jnp.dot(r, w2t_ref[...],
                            preferred_element_type=jnp.float32)
                    + b2_ref[...])


def _final(p, isc, w2t, b2):
    return pl.pallas_call(
        _final_body,
        out_shape=jax.ShapeDtypeStruct((N, C), jnp.float32),
    )(p, isc, w2t, b2)


def kernel(feat, edge_index, W0, W1, W2, b2, g0, bt0, g1, bt1):
    src = edge_index[0]
    dst = edge_index[1]
    z128 = jnp.zeros((NP, H), jnp.float32)
    ones = jnp.ones((KD, H), jnp.float32)

    degs = _degrees(src, dst, z128, ones)
    h0, osc, isc = _pre(feat, degs)
    p = _spmm(h0, src, dst, z128)
    h1 = _post(p, isc, osc, W0.T, g0[None], bt0[None])
    p = _spmm(h1, src, dst, z128)
    h2 = _post(p, isc, osc, W1.T, g1[None], bt1[None])
    p = _spmm(h2, src, dst, z128)
    return _final(p, isc, W2.T, b2[None])


# speedup vs baseline: 1.1135x; 1.0188x over previous
import functools

import jax
import jax.numpy as jnp
from jax import lax
from jax.experimental import pallas as pl
from jax.experimental.pallas import tpu as pltpu
from jax.experimental.pallas import tpu_sc as plsc

N = 10000
E = 320000
D = 128
H = 128
C = 40

NC = 2
NS = 16
NW = NC * NS
EPW = E // NW
KD = 80
EPW2 = E // NS
NCD = EPW2 // KD
NBD = 10
KS = 40
NCS = EPW // KS
NBS = 7
REM = NCS % NBS
NP = 10240
RPT = NP // NS
GR = NP // H

_MESH = dict(core_axis_name="c", subcore_axis_name="s", num_cores=NC,
             num_subcores=NS)


def _deg_body(src_hbm, dst_hbm, z_hbm, ones_hbm, out_hbm,
              e0, e1, e2, e3, e4, e5, e6, e7, e8, e9, ones_v, acc, *sems):
    c = lax.axis_index("c")
    s = lax.axis_index("s")
    row0 = s * RPT
    ebase = s * EPW2
    eidx = (e0, e1, e2, e3, e4, e5, e6, e7, e8, e9)
    dsem = sems[0:NBD]
    ssem = sems[NBD:2 * NBD]
    pltpu.sync_copy(ones_hbm, ones_v)

    def one_count(e_hbm, which):
        pltpu.sync_copy(z_hbm.at[pl.ds(row0, RPT), :],
                        acc.at[pl.ds(row0, RPT), :])

        def start(b, ch):
            pltpu.async_copy(e_hbm.at[pl.ds(ebase + ch * KD, KD)], eidx[b],
                             dsem[b])

        def wait_in(b):
            pltpu.make_async_copy(e_hbm.at[pl.ds(0, KD)], eidx[b],
                                  dsem[b]).wait()

        for b in range(NBD):
            start(b, b)
        plsc.subcore_barrier()

        @pl.loop(0, NCD - NBD, step=NBD)
        def _(g):
            descs = []
            for b in range(NBD):
                wait_in(b)
                descs.append(pltpu.async_copy(ones_v, acc.at[eidx[b]],
                                              ssem[b], add=True))
            for b in range(NBD):
                descs[b].wait()
                start(b, g + NBD + b)

        descs = []
        for b in range(NBD):
            wait_in(b)
            descs.append(pltpu.async_copy(ones_v, acc.at[eidx[b]],
                                          ssem[b], add=True))
        for d in descs:
            d.wait()
        plsc.subcore_barrier()
        pltpu.sync_copy(acc.at[pl.ds(row0, RPT), :],
                        out_hbm.at[which, pl.ds(row0, RPT), :])

    @pl.when(c == 0)
    def _():
        one_count(src_hbm, 0)

    @pl.when(c == 1)
    def _():
        one_count(dst_hbm, 1)


def _degrees(src, dst, z128, ones):
    f = pl.kernel(
        _deg_body,
        out_type=jax.ShapeDtypeStruct((2, NP, H), jnp.float32),
        mesh=plsc.VectorSubcoreMesh(**_MESH),
        scratch_types=(
            [pltpu.VMEM((KD,), jnp.int32) for _ in range(NBD)]
            + [pltpu.VMEM((KD, H), jnp.float32)]
            + [pltpu.VMEM_SHARED((NP, H), jnp.float32)]
            + [pltpu.SemaphoreType.DMA for _ in range(2 * NBD)]
        ),
    )
    return f(src, dst, z128, ones)


def _spmm_body(h_hbm, src_hbm, dst_hbm, z_hbm, out_hbm,
               sidx, d0, d1, d2, d3, d4, d5, d6,
               r0b, r1b, r2b, r3b, r4b, r5b, r6b, acc, *sems):
    c = lax.axis_index("c")
    s = lax.axis_index("s")
    wid = s * NC + c
    ebase = wid * EPW
    row0 = s * RPT
    didx = (d0, d1, d2, d3, d4, d5, d6)
    rows = (r0b, r1b, r2b, r3b, r4b, r5b, r6b)
    dsem = sems[0:NBS]
    gsem = sems[NBS:2 * NBS]
    ssem = sems[2 * NBS:3 * NBS]
    isem = sems[3 * NBS]
    zsem = sems[3 * NBS + 1]

    sdesc = pltpu.async_copy(src_hbm.at[pl.ds(ebase, EPW)], sidx, isem)
    zdesc = pltpu.async_copy(z_hbm.at[pl.ds(row0, RPT), :],
                             acc.at[pl.ds(row0, RPT), :], zsem)
    sdesc.wait()

    def start(b, ch):
        pltpu.async_copy(dst_hbm.at[pl.ds(ebase + ch * KS, KS)], didx[b],
                         dsem[b])
        pltpu.async_copy(h_hbm.at[sidx.at[pl.ds(ch * KS, KS)]], rows[b],
                         gsem[b])

    def wait_in(b):
        pltpu.make_async_copy(dst_hbm.at[pl.ds(0, KS)], didx[b],
                              dsem[b]).wait()
        pltpu.make_async_copy(h_hbm.at[sidx.at[pl.ds(0, KS)]], rows[b],
                              gsem[b]).wait()

    def scatter(b):
        return pltpu.async_copy(rows[b], acc.at[didx[b]], ssem[b],
                                add=True)

    for b in range(NBS):
        start(b, b)
    zdesc.wait()
    plsc.subcore_barrier()

    @pl.loop(0, NCS - REM - NBS, step=NBS)
    def _(g):
        descs = []
        for b in range(NBS):
            wait_in(b)
            descs.append(scatter(b))
        for b in range(NBS):
            descs[b].wait()
            ch2 = g + NBS + b

            @pl.when(ch2 < NCS)
            def _():
                start(b, ch2)

    descs = []
    for b in range(NBS):
        wait_in(b)
        descs.append(scatter(b))
    for b in range(NBS):
        descs[b].wait()
        if b < REM:
            start(b, NCS - REM + b)
    for b in range(REM):
        wait_in(b)
        scatter(b).wait()
    plsc.subcore_barrier()
    pltpu.sync_copy(acc.at[pl.ds(row0, RPT), :],
                    out_hbm.at[c, pl.ds(row0, RPT), :])


def _spmm(h, src, dst, z128):
    f = pl.kernel(
        _spmm_body,
        out_type=jax.ShapeDtypeStruct((NC, NP, H), jnp.float32),
        mesh=plsc.VectorSubcoreMesh(**_MESH),
        scratch_types=(
            [pltpu.VMEM((EPW,), jnp.int32)]
            + [pltpu.VMEM((KS,), jnp.int32) for _ in range(NBS)]
            + [pltpu.VMEM((KS, H), jnp.float32) for _ in range(NBS)]
            + [pltpu.VMEM_SHARED((NP, H), jnp.float32)]
            + [pltpu.SemaphoreType.DMA for _ in range(3 * NBS + 2)]
        ),
    )
    return f(h, src, dst, z128)


def _pre_body(feat_ref, degs_ref, h0_ref, osc_ref, isc_ref):
    scnt = degs_ref[0, 0:N]
    dcnt = degs_ref[1, 0:N]
    osc = lax.rsqrt(jnp.maximum(scnt, 1.0))
    isc = lax.rsqrt(jnp.maximum(dcnt, 1.0))
    h0_ref[...] = feat_ref[...] * osc
    osc_ref[...] = osc
    isc_ref[...] = isc


def _pre(feat, degs):
    return pl.pallas_call(
        _pre_body,
        out_shape=(
            jax.ShapeDtypeStruct((N, D), jnp.float32),
            jax.ShapeDtypeStruct((N, H), jnp.float32),
            jax.ShapeDtypeStruct((N, H), jnp.float32),
        ),
    )(feat, degs)


def _post_body(p_ref, isc_ref, osc_ref, wt_ref, g_ref, bt_ref, out_ref):
    r = (p_ref[0, 0:N] + p_ref[1, 0:N]) * isc_ref[...]
    y = jnp.dot(r, wt_ref[...], preferred_element_type=jnp.float32)
    mu = jnp.mean(y, axis=0, keepdims=True)
    yc = y - mu
    var = jnp.mean(yc * yc, axis=0, keepdims=True)
    h = jnp.maximum(yc * lax.rsqrt(var + 1e-5) * g_ref[...] + bt_ref[...],
                    0.0)
    out_ref[...] = h * osc_ref[...]


def _post(p, isc, osc, wt, g, bt):
    return pl.pallas_call(
        _post_body,
        out_shape=jax.ShapeDtypeStruct((N, H), jnp.float32),
    )(p, isc, osc, wt, g, bt)


def _final_body(p_ref, isc_ref, w2t_ref, b2_ref, out_ref):
    r = (p_ref[0, 0:N] + p_ref[1, 0:N]) * isc_ref[...]
    out_ref[...] = (jnp.dot(r, w2t_ref[...],
                            preferred_element_type=jnp.float32)
                    + b2_ref[...])


def _final(p, isc, w2t, b2):
    return pl.pallas_call(
        _final_body,
        out_shape=jax.ShapeDtypeStruct((N, C), jnp.float32),
    )(p, isc, w2t, b2)


def kernel(feat, edge_index, W0, W1, W2, b2, g0, bt0, g1, bt1):
    src = edge_index[0]
    dst = edge_index[1]
    z128 = jnp.zeros((NP, H), jnp.float32)
    ones = jnp.ones((KD, H), jnp.float32)

    degs = _degrees(src, dst, z128, ones)
    h0, osc, isc = _pre(feat, degs)
    p = _spmm(h0, src, dst, z128)
    h1 = _post(p, isc, osc, W0.T, g0[None], bt0[None])
    p = _spmm(h1, src, dst, z128)
    h2 = _post(p, isc, osc, W1.T, g1[None], bt1[None])
    p = _spmm(h2, src, dst, z128)
    return _final(p, isc, W2.T, b2[None])


# speedup vs baseline: 1.1136x; 1.0001x over previous
import jax
import jax.numpy as jnp
from jax import lax
from jax.experimental import pallas as pl
from jax.experimental.pallas import tpu as pltpu
from jax.experimental.pallas import tpu_sc as plsc

N = 10000
E = 320000
D = 128
H = 128
C = 40

NC = 2
NS = 16
NW = NC * NS
EPW = E // NW
KD = 80
EPW2 = E // NS
NCD = EPW2 // KD
NBD = 10
KS = 40
NCS = EPW // KS
NBS = 7
REM = NCS % NBS
NP = 10240
RPT = NP // NS

_MESH = dict(core_axis_name="c", subcore_axis_name="s", num_cores=NC,
             num_subcores=NS)


def _deg_body(src_hbm, dst_hbm, z_hbm, ones_hbm, out_hbm,
              e0, e1, e2, e3, e4, e5, e6, e7, e8, e9, ones_v, acc, *sems):
    c = lax.axis_index("c")
    s = lax.axis_index("s")
    row0 = s * RPT
    ebase = s * EPW2
    eidx = (e0, e1, e2, e3, e4, e5, e6, e7, e8, e9)
    dsem = sems[0:NBD]
    ssem = sems[NBD:2 * NBD]
    pltpu.sync_copy(ones_hbm, ones_v)

    def one_count(e_hbm, which):
        pltpu.sync_copy(z_hbm.at[pl.ds(row0, RPT), :],
                        acc.at[pl.ds(row0, RPT), :])

        def start(b, ch):
            pltpu.async_copy(e_hbm.at[pl.ds(ebase + ch * KD, KD)], eidx[b],
                             dsem[b])

        def wait_in(b):
            pltpu.make_async_copy(e_hbm.at[pl.ds(0, KD)], eidx[b],
                                  dsem[b]).wait()

        for b in range(NBD):
            start(b, b)
        plsc.subcore_barrier()

        @pl.loop(0, NCD - NBD, step=NBD)
        def _(g):
            descs = []
            for b in range(NBD):
                wait_in(b)
                descs.append(pltpu.async_copy(ones_v, acc.at[eidx[b]],
                                              ssem[b], add=True))
            for b in range(NBD):
                descs[b].wait()
                start(b, g + NBD + b)

        descs = []
        for b in range(NBD):
            wait_in(b)
            descs.append(pltpu.async_copy(ones_v, acc.at[eidx[b]],
                                          ssem[b], add=True))
        for d in descs:
            d.wait()
        plsc.subcore_barrier()
        pltpu.sync_copy(acc.at[pl.ds(row0, RPT), :],
                        out_hbm.at[which, pl.ds(row0, RPT), :])

    @pl.when(c == 0)
    def _():
        one_count(src_hbm, 0)

    @pl.when(c == 1)
    def _():
        one_count(dst_hbm, 1)


def _degrees(src, dst, z128, ones):
    f = pl.kernel(
        _deg_body,
        out_type=jax.ShapeDtypeStruct((2, NP, H), jnp.float32),
        mesh=plsc.VectorSubcoreMesh(**_MESH),
        scratch_types=(
            [pltpu.VMEM((KD,), jnp.int32) for _ in range(NBD)]
            + [pltpu.VMEM((KD, H), jnp.float32)]
            + [pltpu.VMEM_SHARED((NP, H), jnp.float32)]
            + [pltpu.SemaphoreType.DMA for _ in range(2 * NBD)]
        ),
    )
    return f(src, dst, z128, ones)


def _spmm_body(h_hbm, src_hbm, dst_hbm, z_hbm, out_hbm,
               sidx, d0, d1, d2, d3, d4, d5, d6,
               r0b, r1b, r2b, r3b, r4b, r5b, r6b, acc, *sems):
    c = lax.axis_index("c")
    s = lax.axis_index("s")
    wid = s * NC + c
    ebase = wid * EPW
    row0 = s * RPT
    didx = (d0, d1, d2, d3, d4, d5, d6)
    rows = (r0b, r1b, r2b, r3b, r4b, r5b, r6b)
    dsem = sems[0:NBS]
    gsem = sems[NBS:2 * NBS]
    ssem = sems[2 * NBS:3 * NBS]
    isem = sems[3 * NBS]
    zsem = sems[3 * NBS + 1]

    sdesc = pltpu.async_copy(src_hbm.at[pl.ds(ebase, EPW)], sidx, isem)
    zdesc = pltpu.async_copy(z_hbm.at[pl.ds(row0, RPT), :],
                             acc.at[pl.ds(row0, RPT), :], zsem)
    sdesc.wait()

    def start(b, ch):
        pltpu.async_copy(dst_hbm.at[pl.ds(ebase + ch * KS, KS)], didx[b],
                         dsem[b])
        pltpu.async_copy(h_hbm.at[sidx.at[pl.ds(ch * KS, KS)]], rows[b],
                         gsem[b])

    def wait_in(b):
        pltpu.make_async_copy(dst_hbm.at[pl.ds(0, KS)], didx[b],
                              dsem[b]).wait()
        pltpu.make_async_copy(h_hbm.at[sidx.at[pl.ds(0, KS)]], rows[b],
                              gsem[b]).wait()

    def scatter(b):
        return pltpu.async_copy(rows[b], acc.at[didx[b]], ssem[b],
                                add=True)

    for b in range(NBS):
        start(b, b)
    zdesc.wait()
    plsc.subcore_barrier()

    @pl.loop(0, NCS - REM - NBS, step=NBS)
    def _(g):
        descs = []
        for b in range(NBS):
            wait_in(b)
            descs.append(scatter(b))
        for b in range(NBS):
            descs[b].wait()
            ch2 = g + NBS + b

            @pl.when(ch2 < NCS)
            def _():
                start(b, ch2)

    descs = []
    for b in range(NBS):
        wait_in(b)
        descs.append(scatter(b))
    for b in range(NBS):
        descs[b].wait()
        if b < REM:
            start(b, NCS - REM + b)
    for b in range(REM):
        wait_in(b)
        scatter(b).wait()
    plsc.subcore_barrier()
    pltpu.sync_copy(acc.at[pl.ds(row0, RPT), :],
                    out_hbm.at[c, pl.ds(row0, RPT), :])


def _spmm(h, src, dst, z128):
    f = pl.kernel(
        _spmm_body,
        out_type=jax.ShapeDtypeStruct((NC, NP, H), jnp.float32),
        mesh=plsc.VectorSubcoreMesh(**_MESH),
        scratch_types=(
            [pltpu.VMEM((EPW,), jnp.int32)]
            + [pltpu.VMEM((KS,), jnp.int32) for _ in range(NBS)]
            + [pltpu.VMEM((KS, H), jnp.float32) for _ in range(NBS)]
            + [pltpu.VMEM_SHARED((NP, H), jnp.float32)]
            + [pltpu.SemaphoreType.DMA for _ in range(3 * NBS + 2)]
        ),
    )
    return f(h, src, dst, z128)


def _pre_body(feat_ref, degs_ref, h0_ref, osc_ref, isc_ref):
    scnt = degs_ref[0, 0:N]
    dcnt = degs_ref[1, 0:N]
    osc = lax.rsqrt(jnp.maximum(scnt, 1.0))
    isc = lax.rsqrt(jnp.maximum(dcnt, 1.0))
    h0_ref[...] = feat_ref[...] * osc
    osc_ref[...] = osc
    isc_ref[...] = isc


def _pre(feat, degs):
    return pl.pallas_call(
        _pre_body,
        out_shape=(
            jax.ShapeDtypeStruct((N, D), jnp.float32),
            jax.ShapeDtypeStruct((N, H), jnp.float32),
            jax.ShapeDtypeStruct((N, H), jnp.float32),
        ),
    )(feat, degs)


def _post_body(p_ref, isc_ref, osc_ref, wt_ref, g_ref, bt_ref, out_ref):
    r = (p_ref[0, 0:N] + p_ref[1, 0:N]) * isc_ref[...]
    y = jnp.dot(r, wt_ref[...], preferred_element_type=jnp.float32)
    mu = jnp.mean(y, axis=0, keepdims=True)
    yc = y - mu
    var = jnp.mean(yc * yc, axis=0, keepdims=True)
    h = jnp.maximum(yc * lax.rsqrt(var + 1e-5) * g_ref[...] + bt_ref[...],
                    0.0)
    out_ref[...] = h * osc_ref[...]


def _post(p, isc, osc, wt, g, bt):
    return pl.pallas_call(
        _post_body,
        out_shape=jax.ShapeDtypeStruct((N, H), jnp.float32),
    )(p, isc, osc, wt, g, bt)


def _final_body(p_ref, isc_ref, w2t_ref, b2_ref, out_ref):
    r = (p_ref[0, 0:N] + p_ref[1, 0:N]) * isc_ref[...]
    out_ref[...] = (jnp.dot(r, w2t_ref[...],
                            preferred_element_type=jnp.float32)
                    + b2_ref[...])


def _final(p, isc, w2t, b2):
    return pl.pallas_call(
        _final_body,
        out_shape=jax.ShapeDtypeStruct((N, C), jnp.float32),
    )(p, isc, w2t, b2)


def kernel(feat, edge_index, W0, W1, W2, b2, g0, bt0, g1, bt1):
    src = edge_index[0]
    dst = edge_index[1]
    z128 = jnp.zeros((NP, H), jnp.float32)
    ones = jnp.ones((KD, H), jnp.float32)

    degs = _degrees(src, dst, z128, ones)
    h0, osc, isc = _pre(feat, degs)
    p = _spmm(h0, src, dst, z128)
    h1 = _post(p, isc, osc, W0.T, g0[None], bt0[None])
    p = _spmm(h1, src, dst, z128)
    h2 = _post(p, isc, osc, W1.T, g1[None], bt1[None])
    p = _spmm(h2, src, dst, z128)
    return _final(p, isc, W2.T, b2[None])
